# baseline (speedup 1.0000x reference)
import jax
import jax.numpy as jnp
from jax.experimental import pallas as pl
from jax.experimental.pallas import tpu as pltpu

_EPS = 1e-6
_BM = 1024
_BN = 512


def _ln_dense_kernel(x_ref, w_ref, s_ref, b_ref, z_ref, y_ref, ybf_ref):
    n = pl.program_id(1)

    @pl.when(n == 0)
    def _():
        x = x_ref[...]
        mu = jnp.mean(x, axis=-1, keepdims=True)
        xc = x - mu
        var = jnp.mean(xc * xc, axis=-1, keepdims=True)
        y = xc * jax.lax.rsqrt(var + _EPS) * s_ref[...] + b_ref[...]
        y_ref[...] = y
        ybf_ref[...] = y.astype(jnp.bfloat16)

    z_ref[...] = jnp.dot(ybf_ref[...], w_ref[...],
                         preferred_element_type=jnp.float32)


def kernel(x, scale, ln_bias, kernel):
    S, B, H = x.shape
    F = kernel.shape[1]
    M = S * B
    x2 = x.reshape(M, H)
    wbf = kernel.astype(jnp.bfloat16)
    s2 = scale.reshape(1, H)
    b2 = ln_bias.reshape(1, H)

    z, y = pl.pallas_call(
        _ln_dense_kernel,
        grid=(M // _BM, F // _BN),
        in_specs=[
            pl.BlockSpec((_BM, H), lambda i, j: (i, 0)),
            pl.BlockSpec((H, _BN), lambda i, j: (0, j)),
            pl.BlockSpec((1, H), lambda i, j: (0, 0)),
            pl.BlockSpec((1, H), lambda i, j: (0, 0)),
        ],
        out_specs=[
            pl.BlockSpec((_BM, _BN), lambda i, j: (i, j)),
            pl.BlockSpec((_BM, H), lambda i, j: (i, 0)),
        ],
        out_shape=[
            jax.ShapeDtypeStruct((M, F), jnp.float32),
            jax.ShapeDtypeStruct((M, H), jnp.float32),
        ],
        scratch_shapes=[pltpu.VMEM((_BM, H), jnp.bfloat16)],
        compiler_params=pltpu.CompilerParams(
            dimension_semantics=("parallel", "arbitrary"),
        ),
    )(x2, wbf, s2, b2)
    return z.reshape(S, B, F), y.reshape(S, B, H)
